# Initial kernel scaffold; baseline (speedup 1.0000x reference)
import jax
import jax.numpy as jnp
from jax.experimental import pallas as pl


def kernel(inputs, tables, W1, b1, W2, b2):
    raise NotImplementedError("write your pallas kernel here")


# speedup vs baseline: 1.9800x; 1.9800x over previous
import functools

import jax
import jax.numpy as jnp
from jax import lax
from jax.experimental import pallas as pl
from jax.experimental.pallas import tpu as pltpu
from jax.experimental.pallas import tpu_sc as plsc

N_CAT = 26
N_NUM = 13
VOCAB = 100000
DIM = 32
B = 16384
ROWS = B * N_CAT
NC, NS = 2, 16
NW = NC * NS
ROWS_PER_W = ROWS // NW
SL = 128
NSLICE = ROWS_PER_W // SL
GRP = 4
GROUP_ROWS = GRP * SL
NGRP = NSLICE // GRP


def _sc_gather(flat_tables, idx3):
    mesh = plsc.VectorSubcoreMesh(core_axis_name="c", subcore_axis_name="s")

    @functools.partial(
        pl.kernel,
        mesh=mesh,
        compiler_params=pltpu.CompilerParams(use_tc_tiling_on_sc=False),
        out_type=jax.ShapeDtypeStruct((ROWS, DIM), jnp.float32),
        scratch_types=[
            pltpu.VMEM((NSLICE, SL), jnp.int32),
            pltpu.VMEM((GROUP_ROWS, DIM), jnp.float32),
            pltpu.SemaphoreType.DMA,
        ],
    )
    def k(tab_hbm, idx_hbm, out_hbm, idx_v, buf, gsem):
        wid = lax.axis_index("s") * NC + lax.axis_index("c")
        base = wid * ROWS_PER_W
        pltpu.sync_copy(idx_hbm.at[wid], idx_v)

        def body(g, carry):
            handles = [
                pltpu.async_copy(
                    tab_hbm.at[idx_v.at[g * GRP + j]],
                    buf.at[pl.ds(j * SL, SL)],
                    gsem,
                )
                for j in range(GRP)
            ]
            for h in handles:
                h.wait()
            pltpu.sync_copy(buf, out_hbm.at[pl.ds(base + g * GROUP_ROWS, GROUP_ROWS)])
            return carry

        lax.fori_loop(0, NGRP, body, 0)

    return k(flat_tables, idx3)


def _mlp(cat_emb, numeric, W1, b1r, W2, b2r):
    BK = 1024
    IN_DIM = N_CAT * DIM + N_NUM

    def body(cat_ref, num_ref, w1_ref, b1_ref, w2_ref, b2_ref, out_ref):
        x = jnp.concatenate([cat_ref[...], num_ref[...]], axis=1)
        h = jnp.dot(x, w1_ref[...], preferred_element_type=jnp.float32)
        h = jnp.maximum(h + b1_ref[...], 0.0)
        o = jnp.dot(h, w2_ref[...], preferred_element_type=jnp.float32) + b2_ref[0, 0]
        out_ref[...] = 1.0 / (1.0 + jnp.exp(-o))

    return pl.pallas_call(
        body,
        grid=(B // BK,),
        in_specs=[
            pl.BlockSpec((BK, N_CAT * DIM), lambda i: (i, 0)),
            pl.BlockSpec((BK, N_NUM), lambda i: (i, 0)),
            pl.BlockSpec((IN_DIM, 128), lambda i: (0, 0)),
            pl.BlockSpec((1, 128), lambda i: (0, 0)),
            pl.BlockSpec((128, 1), lambda i: (0, 0)),
            pl.BlockSpec((1, 1), lambda i: (0, 0)),
        ],
        out_specs=pl.BlockSpec((BK, 1), lambda i: (i, 0)),
        out_shape=jax.ShapeDtypeStruct((B, 1), jnp.float32),
    )(cat_emb, numeric, W1, b1r, W2, b2r)


def kernel(inputs, tables, W1, b1, W2, b2):
    idx = inputs[:, :N_CAT].astype(jnp.int32)
    flat_idx = idx + (jnp.arange(N_CAT, dtype=jnp.int32) * VOCAB)[None, :]
    idx3 = flat_idx.reshape(NW, NSLICE, SL)
    flat_tables = tables.reshape(N_CAT * VOCAB, DIM)

    emb = _sc_gather(flat_tables, idx3)
    cat_emb = emb.reshape(B, N_CAT * DIM)

    numeric = inputs[:, N_CAT:]
    return _mlp(cat_emb, numeric, W1, b1.reshape(1, 128), W2, b2.reshape(1, 1))


# speedup vs baseline: 2.5183x; 1.2719x over previous
import functools

import jax
import jax.numpy as jnp
from jax import lax
from jax.experimental import pallas as pl
from jax.experimental.pallas import tpu as pltpu
from jax.experimental.pallas import tpu_sc as plsc

N_CAT = 26
N_NUM = 13
VOCAB = 100000
DIM = 32
B = 16384
ROWS = B * N_CAT
NC, NS = 2, 16
NW = NC * NS
ROWS_PER_W = ROWS // NW
SL = 128
NSLICE = ROWS_PER_W // SL
GRP = 4
GROUP_ROWS = GRP * SL
NGRP = NSLICE // GRP

LINES = N_CAT * VOCAB * DIM // 128
VB = 5
VCHUNK = VOCAB // VB
CN, CTAIL = 512, 32
NCH = (VCHUNK - CTAIL) // CN

XT_RB, XT_J, XT_S, XT_L = B // 8, 7, 8, 128
XCHUNKS = XT_RB * XT_J * XT_S * XT_L // DIM


def _relayout_tables(t2):
    def tile(x, n):
        xt = x.T.reshape(n // 4, 4, DIM)
        return jnp.concatenate([xt[:, 0], xt[:, 1], xt[:, 2], xt[:, 3]], axis=1)

    def body(t2_ref, out_ref):
        for k in range(NCH):
            x = t2_ref[0, :, pl.ds(k * CN, CN)]
            out_ref[pl.ds(k * (CN // 4), CN // 4), :] = tile(x, CN)
        x = t2_ref[0, :, pl.ds(NCH * CN, CTAIL)]
        out_ref[pl.ds(NCH * (CN // 4), CTAIL // 4), :] = tile(x, CTAIL)

    return pl.pallas_call(
        body,
        grid=(N_CAT, VB),
        in_specs=[pl.BlockSpec((1, DIM, VOCAB), lambda f, c: (f, 0, c))],
        out_specs=pl.BlockSpec((VCHUNK * DIM // 128, 128),
                               lambda f, c: (f * VB + c, 0)),
        out_shape=jax.ShapeDtypeStruct((LINES, 128), jnp.float32),
    )(t2)


def _sc_gather_scatter(flat_tables, idx3, scat3):
    mesh = plsc.VectorSubcoreMesh(core_axis_name="c", subcore_axis_name="s")

    @functools.partial(
        pl.kernel,
        mesh=mesh,
        compiler_params=pltpu.CompilerParams(use_tc_tiling_on_sc=False),
        out_type=jax.ShapeDtypeStruct((XCHUNKS, DIM), jnp.float32),
        scratch_types=[
            pltpu.VMEM((NSLICE, SL), jnp.int32),
            pltpu.VMEM((NSLICE, SL), jnp.int32),
            pltpu.VMEM((GROUP_ROWS, DIM), jnp.float32),
            pltpu.VMEM((GROUP_ROWS, DIM), jnp.float32),
            pltpu.SemaphoreType.DMA,
            pltpu.SemaphoreType.DMA,
            pltpu.SemaphoreType.DMA,
            pltpu.SemaphoreType.DMA,
        ],
    )
    def k(tab_hbm, idx_hbm, scat_hbm, out_hbm, idx_v, scat_v,
          buf0, buf1, gsem0, gsem1, wsem0, wsem1):
        wid = lax.axis_index("s") * NC + lax.axis_index("c")
        pltpu.sync_copy(idx_hbm.at[wid], idx_v)
        pltpu.sync_copy(scat_hbm.at[wid], scat_v)

        def fire_gather(g, buf, sem):
            for j in range(GRP):
                pltpu.async_copy(
                    tab_hbm.at[idx_v.at[g * GRP + j]],
                    buf.at[pl.ds(j * SL, SL)], sem)

        def drain(buf, sem, n=GRP):
            for j in range(n):
                pltpu.make_async_copy(
                    tab_hbm.at[idx_v.at[0]], buf.at[pl.ds(j * SL, SL)], sem
                ).wait()

        def fire_scatter(g, buf, sem):
            for j in range(GRP):
                pltpu.async_copy(
                    buf.at[pl.ds(j * SL, SL)],
                    out_hbm.at[scat_v.at[g * GRP + j]], sem)

        fire_gather(0, buf0, gsem0)

        def body(g, carry):
            def phase(buf, gsem, wsem, obuf, ogsem):
                drain(buf, gsem)
                @pl.when(g + 1 < NGRP)
                def _():
                    fire_gather(g + 1, obuf, ogsem)
                fire_scatter(g, buf, wsem)
                drain(buf, wsem)

            @pl.when(g % 2 == 0)
            def _():
                phase(buf0, gsem0, wsem0, buf1, gsem1)

            @pl.when(g % 2 == 1)
            def _():
                phase(buf1, gsem1, wsem1, buf0, gsem0)

            return carry

        lax.fori_loop(0, NGRP, body, 0)

    return k(flat_tables, idx3, scat3)


def _mlp(x4, numeric, W1, b1r, W2, b2r):
    BK = 1024
    BKH = BK // 8

    def body(x4_ref, num_ref, w1_ref, b1_ref, w2_ref, b2_ref, out_ref):
        parts = [x4_ref[:, j, :, :].reshape(BK, 128) for j in range(XT_J - 1)]
        parts.append(x4_ref[:, XT_J - 1, :, :].reshape(BK, 128)[:, :64])
        parts.append(num_ref[...])
        x = jnp.concatenate(parts, axis=1)
        h = jnp.dot(x, w1_ref[...], preferred_element_type=jnp.float32)
        h = jnp.maximum(h + b1_ref[...], 0.0)
        o = jnp.dot(h, w2_ref[...], preferred_element_type=jnp.float32) + b2_ref[0, 0]
        out_ref[...] = 1.0 / (1.0 + jnp.exp(-o))

    return pl.pallas_call(
        body,
        grid=(B // BK,),
        in_specs=[
            pl.BlockSpec((BKH, XT_J, XT_S, XT_L), lambda i: (i, 0, 0, 0)),
            pl.BlockSpec((BK, N_NUM), lambda i: (i, 0)),
            pl.BlockSpec((N_CAT * DIM + N_NUM, 128), lambda i: (0, 0)),
            pl.BlockSpec((1, 128), lambda i: (0, 0)),
            pl.BlockSpec((128, 1), lambda i: (0, 0)),
            pl.BlockSpec((1, 1), lambda i: (0, 0)),
        ],
        out_specs=pl.BlockSpec((BK, 1), lambda i: (i, 0)),
        out_shape=jax.ShapeDtypeStruct((B, 1), jnp.float32),
    )(x4, numeric, W1, b1r, W2, b2r)


def kernel(inputs, tables, W1, b1, W2, b2):
    idx = inputs[:, :N_CAT].astype(jnp.int32)
    flat_idx = idx + (jnp.arange(N_CAT, dtype=jnp.int32) * VOCAB)[None, :]
    idx3 = flat_idx.reshape(NW, NSLICE, SL)

    bb = jnp.arange(B, dtype=jnp.int32)[:, None]
    ii = jnp.arange(N_CAT, dtype=jnp.int32)[None, :]
    scat = ((bb // 8) * (XT_J * 32) + (ii // 4) * 32 + (bb % 8) * 4 + (ii % 4))
    scat3 = scat.reshape(NW, NSLICE, SL)

    t2 = jnp.swapaxes(tables, 1, 2)
    tab_lines = _relayout_tables(t2)
    flat_tables = tab_lines.reshape(N_CAT * VOCAB, DIM)

    xflat = _sc_gather_scatter(flat_tables, idx3, scat3)
    x4 = xflat.reshape(XT_RB, XT_J, XT_S, XT_L)

    numeric = inputs[:, N_CAT:]
    return _mlp(x4, numeric, W1, b1.reshape(1, 128), W2, b2.reshape(1, 1))


# speedup vs baseline: 6.5511x; 2.6014x over previous
import functools

import jax
import jax.numpy as jnp
from jax import lax
from jax.experimental import pallas as pl
from jax.experimental.pallas import tpu as pltpu
from jax.experimental.pallas import tpu_sc as plsc

N_CAT = 26
N_NUM = 13
VOCAB = 100000
DIM = 32
B = 16384
ROWS = B * N_CAT
NC, NS = 2, 16
NW = NC * NS
ROWS_PER_W = ROWS // NW
SL = 128
NSLICE = ROWS_PER_W // SL
GRP = 4
GROUP_ROWS = GRP * SL
NGRP = NSLICE // GRP

VBLK = 512
NBLK = 196
LAST_BASE = VOCAB - VBLK
BPG = 28
NGSTEP = NBLK // BPG
LINES = N_CAT * NBLK * 128
TROWS = LINES * 4

XT_RB, XT_J, XT_S, XT_L = B // 8, 7, 8, 128
XCHUNKS = XT_RB * XT_J * XT_S * XT_L // DIM


def _relayout_tables(t2):
    SPAN = BPG * VBLK

    def body(t2_ref, out_ref):
        i = pl.program_id(1)

        @pl.when(i < NGSTEP - 1)
        def _():
            for k in range(BPG):
                x = t2_ref[0, :, pl.ds(k * VBLK, VBLK)]
                s = jnp.concatenate([x[:, 0:128], x[:, 128:256],
                                     x[:, 256:384], x[:, 384:512]], axis=0)
                out_ref[pl.ds(k * 128, 128), :] = s.T

        @pl.when(i == NGSTEP - 1)
        def _():
            for k in range(BPG):
                start = min((NGSTEP - 1) * SPAN + k * VBLK,
                            LAST_BASE) - (NGSTEP - 1) * SPAN
                x = t2_ref[0, :, pl.ds(start, VBLK)]
                s = jnp.concatenate([x[:, 0:128], x[:, 128:256],
                                     x[:, 256:384], x[:, 384:512]], axis=0)
                out_ref[pl.ds(k * 128, 128), :] = s.T

    return pl.pallas_call(
        body,
        grid=(N_CAT, NGSTEP),
        in_specs=[pl.BlockSpec((1, DIM, SPAN), lambda f, i: (f, 0, i))],
        out_specs=pl.BlockSpec((BPG * 128, 128), lambda f, i: (f * NGSTEP + i, 0)),
        out_shape=jax.ShapeDtypeStruct((LINES, 128), jnp.float32),
    )(t2)


def _sc_gather_scatter(flat_tables, idx3, scat3):
    mesh = plsc.VectorSubcoreMesh(core_axis_name="c", subcore_axis_name="s")

    @functools.partial(
        pl.kernel,
        mesh=mesh,
        compiler_params=pltpu.CompilerParams(use_tc_tiling_on_sc=False),
        out_type=jax.ShapeDtypeStruct((XCHUNKS, DIM), jnp.float32),
        scratch_types=[
            pltpu.VMEM((NSLICE, SL), jnp.int32),
            pltpu.VMEM((NSLICE, SL), jnp.int32),
            pltpu.VMEM((GROUP_ROWS, DIM), jnp.float32),
            pltpu.VMEM((GROUP_ROWS, DIM), jnp.float32),
            pltpu.SemaphoreType.DMA,
            pltpu.SemaphoreType.DMA,
            pltpu.SemaphoreType.DMA,
            pltpu.SemaphoreType.DMA,
        ],
    )
    def k(tab_hbm, idx_hbm, scat_hbm, out_hbm, idx_v, scat_v,
          buf0, buf1, gsem0, gsem1, wsem0, wsem1):
        wid = lax.axis_index("s") * NC + lax.axis_index("c")
        pltpu.sync_copy(idx_hbm.at[wid], idx_v)
        pltpu.sync_copy(scat_hbm.at[wid], scat_v)

        def fire_gather(g, buf, sem):
            for j in range(GRP):
                pltpu.async_copy(
                    tab_hbm.at[idx_v.at[g * GRP + j]],
                    buf.at[pl.ds(j * SL, SL)], sem)

        def drain(buf, sem, n=GRP):
            for j in range(n):
                pltpu.make_async_copy(
                    tab_hbm.at[idx_v.at[0]], buf.at[pl.ds(j * SL, SL)], sem
                ).wait()

        def fire_scatter(g, buf, sem):
            for j in range(GRP):
                pltpu.async_copy(
                    buf.at[pl.ds(j * SL, SL)],
                    out_hbm.at[scat_v.at[g * GRP + j]], sem)

        fire_gather(0, buf0, gsem0)

        def body(g, carry):
            def phase(buf, gsem, wsem, obuf, ogsem):
                drain(buf, gsem)
                @pl.when(g + 1 < NGRP)
                def _():
                    fire_gather(g + 1, obuf, ogsem)
                fire_scatter(g, buf, wsem)
                drain(buf, wsem)

            @pl.when(g % 2 == 0)
            def _():
                phase(buf0, gsem0, wsem0, buf1, gsem1)

            @pl.when(g % 2 == 1)
            def _():
                phase(buf1, gsem1, wsem1, buf0, gsem0)

            return carry

        lax.fori_loop(0, NGRP, body, 0)

    return k(flat_tables, idx3, scat3)


def _mlp(x4, numeric, W1, b1r, W2, b2r):
    BK = 1024
    BKH = BK // 8

    def body(x4_ref, num_ref, w1_ref, b1_ref, w2_ref, b2_ref, out_ref):
        parts = [x4_ref[:, j, :, :].reshape(BK, 128) for j in range(XT_J - 1)]
        parts.append(x4_ref[:, XT_J - 1, :, :].reshape(BK, 128)[:, :64])
        parts.append(num_ref[...])
        x = jnp.concatenate(parts, axis=1)
        h = jnp.dot(x, w1_ref[...], preferred_element_type=jnp.float32)
        h = jnp.maximum(h + b1_ref[...], 0.0)
        o = jnp.dot(h, w2_ref[...], preferred_element_type=jnp.float32) + b2_ref[0, 0]
        out_ref[...] = 1.0 / (1.0 + jnp.exp(-o))

    return pl.pallas_call(
        body,
        grid=(B // BK,),
        in_specs=[
            pl.BlockSpec((BKH, XT_J, XT_S, XT_L), lambda i: (i, 0, 0, 0)),
            pl.BlockSpec((BK, N_NUM), lambda i: (i, 0)),
            pl.BlockSpec((N_CAT * DIM + N_NUM, 128), lambda i: (0, 0)),
            pl.BlockSpec((1, 128), lambda i: (0, 0)),
            pl.BlockSpec((128, 1), lambda i: (0, 0)),
            pl.BlockSpec((1, 1), lambda i: (0, 0)),
        ],
        out_specs=pl.BlockSpec((BK, 1), lambda i: (i, 0)),
        out_shape=jax.ShapeDtypeStruct((B, 1), jnp.float32),
    )(x4, numeric, W1, b1r, W2, b2r)


def kernel(inputs, tables, W1, b1, W2, b2):
    idx = inputs[:, :N_CAT].astype(jnp.int32)
    ff = jnp.arange(N_CAT, dtype=jnp.int32)[None, :]
    g = jnp.minimum(idx // VBLK, NBLK - 1)
    r = idx - jnp.minimum(g * VBLK, LAST_BASE)
    flat_idx = ((ff * NBLK + g) * 128 + r % 128) * 4 + r // 128
    idx3 = flat_idx.reshape(NW, NSLICE, SL)

    bb = jnp.arange(B, dtype=jnp.int32)[:, None]
    ii = jnp.arange(N_CAT, dtype=jnp.int32)[None, :]
    scat = ((bb // 8) * (XT_J * 32) + (ii // 4) * 32 + (bb % 8) * 4 + (ii % 4))
    scat3 = scat.reshape(NW, NSLICE, SL)

    t2 = jnp.swapaxes(tables, 1, 2)
    tab_lines = _relayout_tables(t2)
    flat_tables = tab_lines.reshape(TROWS, DIM)

    xflat = _sc_gather_scatter(flat_tables, idx3, scat3)
    x4 = xflat.reshape(XT_RB, XT_J, XT_S, XT_L)

    numeric = inputs[:, N_CAT:]
    return _mlp(x4, numeric, W1, b1.reshape(1, 128), W2, b2.reshape(1, 1))


# speedup vs baseline: 6.5686x; 1.0027x over previous
import functools

import jax
import jax.numpy as jnp
from jax import lax
from jax.experimental import pallas as pl
from jax.experimental.pallas import tpu as pltpu
from jax.experimental.pallas import tpu_sc as plsc

N_CAT = 26
N_NUM = 13
VOCAB = 100000
DIM = 32
B = 16384
ROWS = B * N_CAT
NC, NS = 2, 16
NW = NC * NS
ROWS_PER_W = ROWS // NW
SL = 128
NSLICE = ROWS_PER_W // SL
GRP = 4
GROUP_ROWS = GRP * SL
NGRP = NSLICE // GRP

VBLK = 512
NBLK = 196
LAST_BASE = VOCAB - VBLK
BPG = 28
NGSTEP = NBLK // BPG
LINES = N_CAT * NBLK * 128
TROWS = LINES * 4

XT_RB, XT_J, XT_S, XT_L = B // 8, 7, 8, 128
XCHUNKS = XT_RB * XT_J * XT_S * XT_L // DIM


def _relayout_tables(t2):
    SPAN = BPG * VBLK

    def body(t2_ref, out_ref):
        i = pl.program_id(1)

        @pl.when(i < NGSTEP - 1)
        def _():
            for k in range(BPG):
                x = t2_ref[0, :, pl.ds(k * VBLK, VBLK)]
                s = jnp.concatenate([x[:, 0:128], x[:, 128:256],
                                     x[:, 256:384], x[:, 384:512]], axis=0)
                out_ref[pl.ds(k * 128, 128), :] = s.T

        @pl.when(i == NGSTEP - 1)
        def _():
            for k in range(BPG):
                start = min((NGSTEP - 1) * SPAN + k * VBLK,
                            LAST_BASE) - (NGSTEP - 1) * SPAN
                x = t2_ref[0, :, pl.ds(start, VBLK)]
                s = jnp.concatenate([x[:, 0:128], x[:, 128:256],
                                     x[:, 256:384], x[:, 384:512]], axis=0)
                out_ref[pl.ds(k * 128, 128), :] = s.T

    return pl.pallas_call(
        body,
        grid=(N_CAT, NGSTEP),
        compiler_params=pltpu.CompilerParams(
            dimension_semantics=("parallel", "parallel")),
        in_specs=[pl.BlockSpec((1, DIM, SPAN), lambda f, i: (f, 0, i))],
        out_specs=pl.BlockSpec((BPG * 128, 128), lambda f, i: (f * NGSTEP + i, 0)),
        out_shape=jax.ShapeDtypeStruct((LINES, 128), jnp.float32),
    )(t2)


def _sc_gather_scatter(flat_tables, idx3, scat3):
    mesh = plsc.VectorSubcoreMesh(core_axis_name="c", subcore_axis_name="s")

    @functools.partial(
        pl.kernel,
        mesh=mesh,
        compiler_params=pltpu.CompilerParams(use_tc_tiling_on_sc=False),
        out_type=jax.ShapeDtypeStruct((XCHUNKS, DIM), jnp.float32),
        scratch_types=[
            pltpu.VMEM((NSLICE, SL), jnp.int32),
            pltpu.VMEM((NSLICE, SL), jnp.int32),
            pltpu.VMEM((GROUP_ROWS, DIM), jnp.float32),
            pltpu.VMEM((GROUP_ROWS, DIM), jnp.float32),
            pltpu.SemaphoreType.DMA,
            pltpu.SemaphoreType.DMA,
            pltpu.SemaphoreType.DMA,
            pltpu.SemaphoreType.DMA,
        ],
    )
    def k(tab_hbm, idx_hbm, scat_hbm, out_hbm, idx_v, scat_v,
          buf0, buf1, gsem0, gsem1, wsem0, wsem1):
        wid = lax.axis_index("s") * NC + lax.axis_index("c")
        pltpu.sync_copy(idx_hbm.at[wid], idx_v)
        pltpu.sync_copy(scat_hbm.at[wid], scat_v)

        def fire_gather(g, buf, sem):
            for j in range(GRP):
                pltpu.async_copy(
                    tab_hbm.at[idx_v.at[g * GRP + j]],
                    buf.at[pl.ds(j * SL, SL)], sem)

        def drain(buf, sem, n=GRP):
            for j in range(n):
                pltpu.make_async_copy(
                    tab_hbm.at[idx_v.at[0]], buf.at[pl.ds(j * SL, SL)], sem
                ).wait()

        def fire_scatter(g, buf, sem):
            for j in range(GRP):
                pltpu.async_copy(
                    buf.at[pl.ds(j * SL, SL)],
                    out_hbm.at[scat_v.at[g * GRP + j]], sem)

        fire_gather(0, buf0, gsem0)

        def body(g, carry):
            def phase(buf, gsem, wsem, obuf, ogsem):
                drain(buf, gsem)
                @pl.when(g + 1 < NGRP)
                def _():
                    fire_gather(g + 1, obuf, ogsem)
                fire_scatter(g, buf, wsem)
                drain(buf, wsem)

            @pl.when(g % 2 == 0)
            def _():
                phase(buf0, gsem0, wsem0, buf1, gsem1)

            @pl.when(g % 2 == 1)
            def _():
                phase(buf1, gsem1, wsem1, buf0, gsem0)

            return carry

        lax.fori_loop(0, NGRP, body, 0)

    return k(flat_tables, idx3, scat3)


def _mlp(x4, numeric, W1, b1r, W2, b2r):
    BK = 1024
    BKH = BK // 8

    def body(x4_ref, num_ref, w1_ref, b1_ref, w2_ref, b2_ref, out_ref):
        parts = [x4_ref[:, j, :, :].reshape(BK, 128) for j in range(XT_J - 1)]
        parts.append(x4_ref[:, XT_J - 1, :, :].reshape(BK, 128)[:, :64])
        parts.append(num_ref[...])
        x = jnp.concatenate(parts, axis=1)
        h = jnp.dot(x, w1_ref[...], preferred_element_type=jnp.float32)
        h = jnp.maximum(h + b1_ref[...], 0.0)
        o = jnp.dot(h, w2_ref[...], preferred_element_type=jnp.float32) + b2_ref[0, 0]
        out_ref[...] = 1.0 / (1.0 + jnp.exp(-o))

    return pl.pallas_call(
        body,
        grid=(B // BK,),
        in_specs=[
            pl.BlockSpec((BKH, XT_J, XT_S, XT_L), lambda i: (i, 0, 0, 0)),
            pl.BlockSpec((BK, N_NUM), lambda i: (i, 0)),
            pl.BlockSpec((N_CAT * DIM + N_NUM, 128), lambda i: (0, 0)),
            pl.BlockSpec((1, 128), lambda i: (0, 0)),
            pl.BlockSpec((128, 1), lambda i: (0, 0)),
            pl.BlockSpec((1, 1), lambda i: (0, 0)),
        ],
        out_specs=pl.BlockSpec((BK, 1), lambda i: (i, 0)),
        out_shape=jax.ShapeDtypeStruct((B, 1), jnp.float32),
    )(x4, numeric, W1, b1r, W2, b2r)


def kernel(inputs, tables, W1, b1, W2, b2):
    idx = inputs[:, :N_CAT].astype(jnp.int32)
    ff = jnp.arange(N_CAT, dtype=jnp.int32)[None, :]
    g = jnp.minimum(idx // VBLK, NBLK - 1)
    r = idx - jnp.minimum(g * VBLK, LAST_BASE)
    flat_idx = ((ff * NBLK + g) * 128 + r % 128) * 4 + r // 128
    idx3 = flat_idx.reshape(NW, NSLICE, SL)

    bb = jnp.arange(B, dtype=jnp.int32)[:, None]
    ii = jnp.arange(N_CAT, dtype=jnp.int32)[None, :]
    scat = ((bb // 8) * (XT_J * 32) + (ii // 4) * 32 + (bb % 8) * 4 + (ii % 4))
    scat3 = scat.reshape(NW, NSLICE, SL)

    t2 = jnp.swapaxes(tables, 1, 2)
    tab_lines = _relayout_tables(t2)
    flat_tables = tab_lines.reshape(TROWS, DIM)

    xflat = _sc_gather_scatter(flat_tables, idx3, scat3)
    x4 = xflat.reshape(XT_RB, XT_J, XT_S, XT_L)

    numeric = inputs[:, N_CAT:]
    return _mlp(x4, numeric, W1, b1.reshape(1, 128), W2, b2.reshape(1, 1))
